# baseline (speedup 1.0000x reference)
import functools

import jax
import jax.numpy as jnp
from jax import lax
from jax.experimental import pallas as pl
from jax.experimental.pallas import tpu as pltpu
from jax.experimental.pallas import tpu_sc as plsc

_INFO = plsc.get_sparse_core_info()
_NC, _NS = _INFO.num_cores, _INFO.num_subcores
_NW = _NC * _NS


def kernel(x, pos_emb):
    S = x.shape[0]
    D = pos_emb.shape[1]
    src = pos_emb[:S]
    rows_per_w = S // _NW
    mesh = plsc.VectorSubcoreMesh(core_axis_name="c", subcore_axis_name="s")

    @functools.partial(
        pl.kernel,
        out_type=jax.ShapeDtypeStruct((S, D), jnp.float32),
        mesh=mesh,
    )
    def _copy(src_hbm, out_hbm):
        wid = lax.axis_index("s") * _NC + lax.axis_index("c")
        base = wid * rows_per_w
        pltpu.sync_copy(
            src_hbm.at[pl.ds(base, rows_per_w)],
            out_hbm.at[pl.ds(base, rows_per_w)],
        )

    return _copy(src).reshape(S, 1, D)


# speedup vs baseline: 14.6313x; 14.6313x over previous
import functools

import jax
import jax.numpy as jnp
from jax import lax
from jax.experimental import pallas as pl
from jax.experimental.pallas import tpu as pltpu
from jax.experimental.pallas import tpu_sc as plsc

_INFO = plsc.get_sparse_core_info()
_NC, _NS = _INFO.num_cores, _INFO.num_subcores
_NW = _NC * _NS
_CHUNK = 32


def kernel(x, pos_emb):
    S = x.shape[0]
    D = pos_emb.shape[1]
    src = pos_emb[:S]
    rows_per_w = S // _NW
    nchunks = rows_per_w // _CHUNK
    mesh = plsc.VectorSubcoreMesh(core_axis_name="c", subcore_axis_name="s")

    @functools.partial(
        pl.kernel,
        out_type=jax.ShapeDtypeStruct((S, D), jnp.float32),
        mesh=mesh,
        scratch_types=[
            pltpu.VMEM((2, _CHUNK, D), jnp.float32),
            pltpu.SemaphoreType.DMA((2,)),
            pltpu.SemaphoreType.DMA((2,)),
        ],
    )
    def _copy(src_hbm, out_hbm, buf, rsem, wsem):
        wid = lax.axis_index("s") * _NC + lax.axis_index("c")
        base = wid * rows_per_w

        def read(i):
            return pltpu.make_async_copy(
                src_hbm.at[pl.ds(base + i * _CHUNK, _CHUNK)],
                buf.at[i % 2],
                rsem.at[i % 2],
            )

        def write(i):
            return pltpu.make_async_copy(
                buf.at[i % 2],
                out_hbm.at[pl.ds(base + i * _CHUNK, _CHUNK)],
                wsem.at[i % 2],
            )

        read(0).start()
        for i in range(nchunks):
            read(i).wait()
            write(i).start()
            if i + 1 < nchunks:
                if i >= 1:
                    write(i - 1).wait()
                read(i + 1).start()
        if nchunks >= 2:
            write(nchunks - 2).wait()
        write(nchunks - 1).wait()

    return _copy(src).reshape(S, 1, D)


# speedup vs baseline: 22.4809x; 1.5365x over previous
import functools

import jax
import jax.numpy as jnp
from jax import lax
from jax.experimental import pallas as pl
from jax.experimental.pallas import tpu as pltpu
from jax.experimental.pallas import tpu_sc as plsc

_INFO = plsc.get_sparse_core_info()
_NC, _NS = _INFO.num_cores, _INFO.num_subcores
_NW = _NC * _NS
_CHUNK = 32


def kernel(x, pos_emb):
    S = x.shape[0]
    D = pos_emb.shape[1]
    src = pos_emb[:S]
    rows_per_w = S // _NW
    nchunks = rows_per_w // _CHUNK
    mesh = plsc.VectorSubcoreMesh(core_axis_name="c", subcore_axis_name="s")

    @functools.partial(
        pl.kernel,
        out_type=jax.ShapeDtypeStruct((S, 1, D), jnp.float32),
        mesh=mesh,
        scratch_types=[
            pltpu.VMEM((2, _CHUNK, D), jnp.float32),
            pltpu.SemaphoreType.DMA((2,)),
            pltpu.SemaphoreType.DMA((2,)),
        ],
    )
    def _copy(src_hbm, out_hbm, buf, rsem, wsem):
        wid = lax.axis_index("s") * _NC + lax.axis_index("c")
        base = wid * rows_per_w

        def read(i):
            return pltpu.make_async_copy(
                src_hbm.at[pl.ds(base + i * _CHUNK, _CHUNK)],
                buf.at[i % 2],
                rsem.at[i % 2],
            )

        def write(i):
            return pltpu.make_async_copy(
                buf.at[i % 2],
                out_hbm.at[pl.ds(base + i * _CHUNK, _CHUNK), 0],
                wsem.at[i % 2],
            )

        read(0).start()
        for i in range(nchunks):
            read(i).wait()
            write(i).start()
            if i + 1 < nchunks:
                if i >= 1:
                    write(i - 1).wait()
                read(i + 1).start()
        if nchunks >= 2:
            write(nchunks - 2).wait()
        write(nchunks - 1).wait()

    return _copy(src)


# speedup vs baseline: 25.1983x; 1.1209x over previous
import functools

import jax
import jax.numpy as jnp
from jax import lax
from jax.experimental import pallas as pl
from jax.experimental.pallas import tpu as pltpu
from jax.experimental.pallas import tpu_sc as plsc

_INFO = plsc.get_sparse_core_info()
_NC, _NS = _INFO.num_cores, _INFO.num_subcores
_NW = _NC * _NS
_CHUNK = 32


def kernel(x, pos_emb):
    S = x.shape[0]
    D = pos_emb.shape[1]
    src = pos_emb[:S]
    rows_per_w = S // _NW
    nchunks = rows_per_w // _CHUNK
    mesh = plsc.VectorSubcoreMesh(core_axis_name="c", subcore_axis_name="s")

    @functools.partial(
        pl.kernel,
        out_type=jax.ShapeDtypeStruct((S, 1, D), jnp.float32),
        mesh=mesh,
        scratch_types=[
            pltpu.VMEM((3, _CHUNK, D), jnp.float32),
            pltpu.SemaphoreType.DMA((3,)),
            pltpu.SemaphoreType.DMA((3,)),
        ],
    )
    def _copy(src_hbm, out_hbm, buf, rsem, wsem):
        wid = lax.axis_index("s") * _NC + lax.axis_index("c")
        base = wid * rows_per_w

        def read(i):
            return pltpu.make_async_copy(
                src_hbm.at[pl.ds(base + i * _CHUNK, _CHUNK)],
                buf.at[i % 3],
                rsem.at[i % 3],
            )

        def write(i):
            return pltpu.make_async_copy(
                buf.at[i % 3],
                out_hbm.at[pl.ds(base + i * _CHUNK, _CHUNK), 0],
                wsem.at[i % 3],
            )

        read(0).start()
        read(1).start()
        for i in range(nchunks):
            read(i).wait()
            write(i).start()
            if i + 2 < nchunks:
                if i >= 1:
                    write(i - 1).wait()
                read(i + 2).start()
        for j in (nchunks - 3, nchunks - 2, nchunks - 1):
            if j >= 0:
                write(j).wait()

    return _copy(src)
